# baseline (speedup 1.0000x reference)
import jax
import jax.numpy as jnp
from jax.experimental import pallas as pl
from jax.experimental.pallas import tpu as pltpu

_OUT_DIM = 4
_SLAB = 128


def _mlp_kernel(x_ref, w1_ref, b1_ref, w2_ref, b2_ref, o_ref):
    tbs = x_ref.shape[0]
    x = x_ref[...].reshape(tbs * _SLAB, x_ref.shape[2])
    h = jnp.dot(x, w1_ref[...], preferred_element_type=jnp.float32)
    h = jnp.maximum(h + b1_ref[...], 0.0)
    logits = jnp.dot(h, w2_ref[...], preferred_element_type=jnp.float32)
    o_ref[...] = (logits + b2_ref[...]).astype(o_ref.dtype)


def kernel(x, w1p, b1p, w2p, b2p):
    B, in_dim = x.shape
    w2s = w2p[:, :_OUT_DIM]
    b2s = b2p[:, :_OUT_DIM]

    xv = jnp.reshape(x, (B // _SLAB, _SLAB, in_dim))
    n_slabs = xv.shape[0]
    tbs = 128
    n_tiles = n_slabs // tbs
    tb = tbs * _SLAB

    out = pl.pallas_call(
        _mlp_kernel,
        out_shape=jax.ShapeDtypeStruct((B, _OUT_DIM), jnp.float32),
        grid=(n_tiles,),
        in_specs=[
            pl.BlockSpec((tbs, _SLAB, in_dim), lambda i: (i, 0, 0)),
            pl.BlockSpec(w1p.shape, lambda i: (0, 0)),
            pl.BlockSpec(b1p.shape, lambda i: (0, 0)),
            pl.BlockSpec(w2s.shape, lambda i: (0, 0)),
            pl.BlockSpec(b2s.shape, lambda i: (0, 0)),
        ],
        out_specs=pl.BlockSpec((tb, _OUT_DIM), lambda i: (i, 0)),
        compiler_params=pltpu.CompilerParams(
            dimension_semantics=("parallel",)
        ),
    )(xv, w1p, b1p, w2s, b2s)

    return out


# speedup vs baseline: 1.1607x; 1.1607x over previous
import jax
import jax.numpy as jnp
from jax.experimental import pallas as pl
from jax.experimental.pallas import tpu as pltpu

_OUT_DIM = 4
_SLAB = 128
_NSTREAM = 4


def _mlp_kernel(x0, x1, x2, x3, w1_ref, b1_ref, w2_ref, b2_ref, o_ref):
    tbs = x0.shape[0]
    w1 = w1_ref[...]
    b1 = b1_ref[...]
    w2 = w2_ref[...]
    b2 = b2_ref[...]
    for s, xs in enumerate((x0, x1, x2, x3)):
        x = xs[...].reshape(tbs * _SLAB, xs.shape[2])
        h = jnp.maximum(jnp.dot(x, w1, preferred_element_type=jnp.float32) + b1, 0.0)
        logits = jnp.dot(h, w2, preferred_element_type=jnp.float32) + b2
        o_ref[s * tbs:(s + 1) * tbs, :, :] = (
            logits.reshape(tbs, _SLAB, _OUT_DIM).astype(o_ref.dtype))


def kernel(x, w1p, b1p, w2p, b2p):
    B, in_dim = x.shape
    w2s = w2p[:, :_OUT_DIM]
    b2s = b2p[:, :_OUT_DIM]

    xv = jnp.reshape(x, (B // _SLAB, _SLAB, in_dim))
    n_slabs = xv.shape[0]
    tbs = 32
    step = _NSTREAM * tbs
    n_tiles = n_slabs // step

    def in_map(s):
        return lambda i: (_NSTREAM * i + s, 0, 0)

    out = pl.pallas_call(
        _mlp_kernel,
        out_shape=jax.ShapeDtypeStruct((n_slabs, _SLAB, _OUT_DIM), jnp.float32),
        grid=(n_tiles,),
        in_specs=[
            pl.BlockSpec((tbs, _SLAB, in_dim), in_map(0)),
            pl.BlockSpec((tbs, _SLAB, in_dim), in_map(1)),
            pl.BlockSpec((tbs, _SLAB, in_dim), in_map(2)),
            pl.BlockSpec((tbs, _SLAB, in_dim), in_map(3)),
            pl.BlockSpec(w1p.shape, lambda i: (0, 0)),
            pl.BlockSpec(b1p.shape, lambda i: (0, 0)),
            pl.BlockSpec(w2s.shape, lambda i: (0, 0)),
            pl.BlockSpec(b2s.shape, lambda i: (0, 0)),
        ],
        out_specs=pl.BlockSpec((step, _SLAB, _OUT_DIM), lambda i: (i, 0, 0)),
        compiler_params=pltpu.CompilerParams(
            dimension_semantics=("parallel",)
        ),
    )(xv, xv, xv, xv, w1p, b1p, w2s, b2s)

    return jnp.reshape(out, (B, _OUT_DIM))


# speedup vs baseline: 1.1619x; 1.0011x over previous
import jax
import jax.numpy as jnp
from jax.experimental import pallas as pl
from jax.experimental.pallas import tpu as pltpu

_OUT_DIM = 4
_SLAB = 128


def _mlp_kernel(x_ref, w1_ref, b1_ref, w2_ref, b2_ref, o_ref):
    tbs = x_ref.shape[0]
    x = x_ref[...].reshape(tbs * _SLAB, x_ref.shape[2])
    h = jnp.dot(x, w1_ref[...], preferred_element_type=jnp.float32)
    h = jnp.maximum(h + b1_ref[...], 0.0)
    logits = jnp.dot(h, w2_ref[...], preferred_element_type=jnp.float32)
    logits = logits + b2_ref[...]
    o_ref[...] = logits.reshape(tbs, _SLAB, _OUT_DIM).astype(o_ref.dtype)


def kernel(x, w1p, b1p, w2p, b2p):
    B, in_dim = x.shape
    w2s = w2p[:, :_OUT_DIM]
    b2s = b2p[:, :_OUT_DIM]

    xv = jnp.reshape(x, (B // _SLAB, _SLAB, in_dim))
    n_slabs = xv.shape[0]
    tbs = 128
    n_tiles = n_slabs // tbs

    out = pl.pallas_call(
        _mlp_kernel,
        out_shape=jax.ShapeDtypeStruct((n_slabs, _SLAB, _OUT_DIM), jnp.float32),
        grid=(n_tiles,),
        in_specs=[
            pl.BlockSpec((tbs, _SLAB, in_dim), lambda i: (i, 0, 0)),
            pl.BlockSpec(w1p.shape, lambda i: (0, 0)),
            pl.BlockSpec(b1p.shape, lambda i: (0, 0)),
            pl.BlockSpec(w2s.shape, lambda i: (0, 0)),
            pl.BlockSpec(b2s.shape, lambda i: (0, 0)),
        ],
        out_specs=pl.BlockSpec((tbs, _SLAB, _OUT_DIM), lambda i: (i, 0, 0)),
        compiler_params=pltpu.CompilerParams(
            dimension_semantics=("parallel",)
        ),
    )(xv, w1p, b1p, w2s, b2s)

    return jnp.reshape(out, (B, _OUT_DIM))
